# Initial kernel scaffold; baseline (speedup 1.0000x reference)
import jax
import jax.numpy as jnp
from jax.experimental import pallas as pl


def kernel(input, locations, lin_w, lin_b, node_emb, edge_emb, hgcn_w, hgcn_b):
    raise NotImplementedError("write your pallas kernel here")


# speedup vs baseline: 61.2625x; 61.2625x over previous
import jax
import jax.numpy as jnp
from jax.experimental import pallas as pl

_TOPK = 8


def _stage1(ne_ref, ee_ref, inpx_ref, wf_ref, adet_ref):
    j = pl.program_id(0)
    nb = ne_ref[...]
    ee = ee_ref[...]
    logits = jax.lax.dot_general(nb, ee, (((1,), (1,)), ((), ())),
                                 preferred_element_type=jnp.float32)
    logits = jnp.maximum(logits, 0.0)
    R, E = logits.shape
    iota = jax.lax.broadcasted_iota(jnp.int32, (R, E), 1)
    mx0 = jnp.max(logits, axis=-1, keepdims=True)
    work = logits
    sel = jnp.zeros((R, E), jnp.bool_)
    for k in range(_TOPK):
        mx = mx0 if k == 0 else jnp.max(work, axis=-1, keepdims=True)
        ismx = work == mx
        first = jnp.min(jnp.where(ismx, iota, E), axis=-1, keepdims=True)
        pick = iota == first
        sel = jnp.logical_or(sel, pick)
        work = jnp.where(pick, -jnp.inf, work)
    ex = jnp.where(sel, jnp.exp(logits - mx0), 0.0)
    wf = ex / jnp.sum(ex, axis=-1, keepdims=True)
    wf_ref[...] = wf
    inpx = inpx_ref[0]
    part = jax.lax.dot_general(wf, inpx, (((0,), (1,)), ((), ())),
                               preferred_element_type=jnp.float32,
                               precision=jax.lax.Precision.HIGHEST)

    @pl.when(j == 0)
    def _():
        adet_ref[...] = jnp.zeros_like(adet_ref)

    adet_ref[...] += part


def _stage2(wf_ref, adet_ref, lw_ref, lb_ref, hw_ref, hb_ref, out_ref):
    wf = wf_ref[...]
    adet = adet_ref[...]
    de = adet[:, 8:9]
    m = adet / jnp.maximum(de, 1e-6)
    ab = jax.lax.dot_general(wf, m, (((1,), (0,)), ((), ())),
                             preferred_element_type=jnp.float32,
                             precision=jax.lax.Precision.HIGHEST)
    dv = jnp.maximum(jnp.sum(wf, axis=-1, keepdims=True), 1e-6)
    u = jax.lax.dot_general(lw_ref[...], hw_ref[...], (((1,), (0,)), ((), ())),
                            preferred_element_type=jnp.float32,
                            precision=jax.lax.Precision.HIGHEST)
    v = jax.lax.dot_general(lb_ref[...], hw_ref[...], (((1,), (0,)), ((), ())),
                            preferred_element_type=jnp.float32,
                            precision=jax.lax.Precision.HIGHEST)
    hb = hb_ref[...]
    base = (ab[:, 8:9] / dv) * v + hb
    for bs in range(8):
        x = (ab[:, bs:bs + 1] / dv) * u + base
        out_ref[bs, :, :] = jnp.where(x > 0, x, jnp.exp(x) - 1.0)


def kernel(input, locations, lin_w, lin_b, node_emb, edge_emb, hgcn_w, hgcn_b):
    del locations
    B, S, N, _ = input.shape
    E, D = edge_emb.shape
    BS = B * S
    R = 1000
    nblk = N // R
    inp8 = input.reshape(BS, N)
    inpx = jnp.concatenate(
        [inp8, jnp.ones((1, N), jnp.float32), jnp.zeros((16 - BS - 1, N), jnp.float32)],
        axis=0)
    inpx = inpx.reshape(16, nblk, R).swapaxes(0, 1)

    wf, adet = pl.pallas_call(
        _stage1,
        grid=(nblk,),
        in_specs=[
            pl.BlockSpec((R, D), lambda j: (j, 0)),
            pl.BlockSpec((E, D), lambda j: (0, 0)),
            pl.BlockSpec((1, 16, R), lambda j: (j, 0, 0)),
        ],
        out_specs=[
            pl.BlockSpec((R, E), lambda j: (j, 0)),
            pl.BlockSpec((E, 16), lambda j: (0, 0)),
        ],
        out_shape=[
            jax.ShapeDtypeStruct((N, E), jnp.float32),
            jax.ShapeDtypeStruct((E, 16), jnp.float32),
        ],
    )(node_emb, edge_emb, inpx)

    out = pl.pallas_call(
        _stage2,
        grid=(nblk,),
        in_specs=[
            pl.BlockSpec((R, E), lambda j: (j, 0)),
            pl.BlockSpec((E, 16), lambda j: (0, 0)),
            pl.BlockSpec((1, D), lambda j: (0, 0)),
            pl.BlockSpec((1, D), lambda j: (0, 0)),
            pl.BlockSpec((D, D), lambda j: (0, 0)),
            pl.BlockSpec((1, D), lambda j: (0, 0)),
        ],
        out_specs=pl.BlockSpec((BS, R, D), lambda j: (0, j, 0)),
        out_shape=jax.ShapeDtypeStruct((BS, N, D), jnp.float32),
    )(wf, adet, lin_w, lin_b.reshape(1, D), hgcn_w, hgcn_b.reshape(1, D))

    return out.reshape(B, S, N, D)


# speedup vs baseline: 65.5163x; 1.0694x over previous
import jax
import jax.numpy as jnp
from jax.experimental import pallas as pl

_TOPK = 8


def _stage1(ne_ref, ee_ref, inpx_ref, wf_ref, adet_ref):
    nb = ne_ref[...]
    ee = ee_ref[...]
    logits = jax.lax.dot_general(nb, ee, (((1,), (1,)), ((), ())),
                                 preferred_element_type=jnp.float32)
    logits = jnp.maximum(logits, 0.0)
    R, E = logits.shape
    iota = jax.lax.broadcasted_iota(jnp.int32, (R, E), 1)
    mx0 = jnp.max(logits, axis=-1, keepdims=True)
    work = logits
    sel = jnp.zeros((R, E), jnp.bool_)
    for k in range(_TOPK):
        mx = mx0 if k == 0 else jnp.max(work, axis=-1, keepdims=True)
        ismx = work == mx
        first = jnp.min(jnp.where(ismx, iota, E), axis=-1, keepdims=True)
        pick = iota == first
        sel = jnp.logical_or(sel, pick)
        work = jnp.where(pick, -jnp.inf, work)
    ex = jnp.where(sel, jnp.exp(logits - mx0), 0.0)
    wf = ex / jnp.sum(ex, axis=-1, keepdims=True)
    wf_ref[...] = wf
    inpx = inpx_ref[...]
    adet_ref[...] = jax.lax.dot_general(
        wf, inpx, (((0,), (1,)), ((), ())),
        preferred_element_type=jnp.float32,
        precision=jax.lax.Precision.HIGHEST)


def _stage2(wf_ref, adet_ref, lw_ref, lb_ref, hw_ref, hb_ref, out_ref):
    wf = wf_ref[...]
    adet = adet_ref[...]
    de = adet[:, 8:9]
    m = adet / jnp.maximum(de, 1e-6)
    ab = jax.lax.dot_general(wf, m, (((1,), (0,)), ((), ())),
                             preferred_element_type=jnp.float32,
                             precision=jax.lax.Precision.HIGHEST)
    dv = jnp.maximum(jnp.sum(wf, axis=-1, keepdims=True), 1e-6)
    u = jax.lax.dot_general(lw_ref[...], hw_ref[...], (((1,), (0,)), ((), ())),
                            preferred_element_type=jnp.float32,
                            precision=jax.lax.Precision.HIGHEST)
    v = jax.lax.dot_general(lb_ref[...], hw_ref[...], (((1,), (0,)), ((), ())),
                            preferred_element_type=jnp.float32,
                            precision=jax.lax.Precision.HIGHEST)
    hb = hb_ref[...]
    base = (ab[:, 8:9] / dv) * v + hb
    for bs in range(8):
        x = (ab[:, bs:bs + 1] / dv) * u + base
        out_ref[bs, :, :] = jnp.where(x > 0, x, jnp.exp(x) - 1.0)


def kernel(input, locations, lin_w, lin_b, node_emb, edge_emb, hgcn_w, hgcn_b):
    del locations
    B, S, N, _ = input.shape
    E, D = edge_emb.shape
    BS = B * S
    R = 1000
    nblk = N // R
    inp8 = input.reshape(BS, N)
    inpx = jnp.concatenate(
        [inp8, jnp.ones((1, N), jnp.float32), jnp.zeros((16 - BS - 1, N), jnp.float32)],
        axis=0)

    wf, adet = pl.pallas_call(
        _stage1,
        out_shape=[
            jax.ShapeDtypeStruct((N, E), jnp.float32),
            jax.ShapeDtypeStruct((E, 16), jnp.float32),
        ],
    )(node_emb, edge_emb, inpx)

    out = pl.pallas_call(
        _stage2,
        grid=(nblk,),
        in_specs=[
            pl.BlockSpec((R, E), lambda j: (j, 0)),
            pl.BlockSpec((E, 16), lambda j: (0, 0)),
            pl.BlockSpec((1, D), lambda j: (0, 0)),
            pl.BlockSpec((1, D), lambda j: (0, 0)),
            pl.BlockSpec((D, D), lambda j: (0, 0)),
            pl.BlockSpec((1, D), lambda j: (0, 0)),
        ],
        out_specs=pl.BlockSpec((BS, R, D), lambda j: (0, j, 0)),
        out_shape=jax.ShapeDtypeStruct((BS, N, D), jnp.float32),
    )(wf, adet, lin_w, lin_b.reshape(1, D), hgcn_w, hgcn_b.reshape(1, D))

    return out.reshape(B, S, N, D)
